# Initial kernel scaffold; baseline (speedup 1.0000x reference)
import functools

import jax
import jax.numpy as jnp
from jax import lax
from jax.experimental import pallas as pl
from jax.experimental.pallas import tpu as pltpu
from jax.experimental.pallas import tpu_sc as plsc

VOCAB = 1000000
EMB = 64
BATCH = 16384
HIST = 50

B = BATCH * HIST
IDX_W = 128
NW = 32
ROWS_PER_W = B // (NW * IDX_W)
STREAMS_PER_CHUNK = 5
CHUNK = STREAMS_PER_CHUNK * IDX_W
CHUNKS = ROWS_PER_W // STREAMS_PER_CHUNK


def _make_kernel():
    mesh = plsc.VectorSubcoreMesh(core_axis_name="c", subcore_axis_name="s")

    @functools.partial(
        pl.kernel,
        mesh=mesh,
        out_type=jax.ShapeDtypeStruct((B, EMB), jnp.float32),
        scratch_types=[
            pltpu.VMEM((ROWS_PER_W, IDX_W), jnp.int32),
            pltpu.VMEM((CHUNK, EMB), jnp.float32),
            pltpu.SemaphoreType.DMA,
        ],
    )
    def emb_kernel(items_hbm, table_hbm, out_hbm, idx_v, rows_v, gsem):
        wid = lax.axis_index("s") * 2 + lax.axis_index("c")
        row_base = wid * ROWS_PER_W
        pltpu.sync_copy(items_hbm.at[pl.ds(row_base, ROWS_PER_W)], idx_v)

        def chunk_body(g):
            handles = []
            for j in range(STREAMS_PER_CHUNK):
                h = pltpu.async_copy(
                    table_hbm.at[idx_v.at[g * STREAMS_PER_CHUNK + j]],
                    rows_v.at[pl.ds(j * IDX_W, IDX_W)],
                    gsem,
                )
                handles.append(h)
            for h in handles:
                h.wait()
            out_base = row_base * IDX_W + g * CHUNK
            pltpu.sync_copy(rows_v, out_hbm.at[pl.ds(out_base, CHUNK)])

        pl.loop(0, CHUNKS)(chunk_body)

    return emb_kernel


_emb_kernel = _make_kernel()


def kernel(items, weight):
    items2d = items.reshape(B // IDX_W, IDX_W).astype(jnp.int32)
    out = _emb_kernel(items2d, weight)
    return out.reshape(BATCH, HIST, EMB)


# speedup vs baseline: 1.8443x; 1.8443x over previous
import functools

import jax
import jax.numpy as jnp
from jax import lax
from jax.experimental import pallas as pl
from jax.experimental.pallas import tpu as pltpu
from jax.experimental.pallas import tpu_sc as plsc

VOCAB = 1000000
EMB = 64
BATCH = 16384
HIST = 50

B = BATCH * HIST
IDX_W = 128
NW = 32
ROWS_PER_W = B // (NW * IDX_W)
STREAMS_PER_CHUNK = 5
CHUNK = STREAMS_PER_CHUNK * IDX_W
CHUNKS = ROWS_PER_W // STREAMS_PER_CHUNK


def _make_kernel():
    mesh = plsc.VectorSubcoreMesh(core_axis_name="c", subcore_axis_name="s")

    @functools.partial(
        pl.kernel,
        mesh=mesh,
        compiler_params=pltpu.CompilerParams(use_tc_tiling_on_sc=False),
        out_type=jax.ShapeDtypeStruct((B, EMB), jnp.float32),
        scratch_types=[
            pltpu.VMEM((ROWS_PER_W, IDX_W), jnp.int32),
            pltpu.VMEM((CHUNK, EMB), jnp.float32),
            pltpu.SemaphoreType.DMA,
        ],
    )
    def emb_kernel(items_hbm, table_hbm, out_hbm, idx_v, rows_v, gsem):
        wid = lax.axis_index("s") * 2 + lax.axis_index("c")
        row_base = wid * ROWS_PER_W
        pltpu.sync_copy(items_hbm.at[pl.ds(row_base, ROWS_PER_W)], idx_v)

        def chunk_body(g):
            handles = []
            for j in range(STREAMS_PER_CHUNK):
                h = pltpu.async_copy(
                    table_hbm.at[idx_v.at[g * STREAMS_PER_CHUNK + j]],
                    rows_v.at[pl.ds(j * IDX_W, IDX_W)],
                    gsem,
                )
                handles.append(h)
            for h in handles:
                h.wait()
            out_base = row_base * IDX_W + g * CHUNK
            pltpu.sync_copy(rows_v, out_hbm.at[pl.ds(out_base, CHUNK)])

        pl.loop(0, CHUNKS)(chunk_body)

    return emb_kernel


_emb_kernel = _make_kernel()


def kernel(items, weight):
    items2d = items.reshape(B // IDX_W, IDX_W).astype(jnp.int32)
    out = _emb_kernel(items2d, weight)
    return out.reshape(BATCH, HIST, EMB)


# speedup vs baseline: 1.8724x; 1.0152x over previous
import functools

import jax
import jax.numpy as jnp
from jax import lax
from jax.experimental import pallas as pl
from jax.experimental.pallas import tpu as pltpu
from jax.experimental.pallas import tpu_sc as plsc

VOCAB = 1000000
EMB = 64
BATCH = 16384
HIST = 50

B = BATCH * HIST
IDX_W = 128
NW = 32
ROWS_PER_W = B // (NW * IDX_W)
STREAMS_PER_CHUNK = 5
CHUNK = STREAMS_PER_CHUNK * IDX_W
CHUNKS = ROWS_PER_W // STREAMS_PER_CHUNK


def _make_kernel():
    mesh = plsc.VectorSubcoreMesh(core_axis_name="c", subcore_axis_name="s")

    @functools.partial(
        pl.kernel,
        mesh=mesh,
        compiler_params=pltpu.CompilerParams(use_tc_tiling_on_sc=False),
        out_type=jax.ShapeDtypeStruct((B, EMB), jnp.float32),
        scratch_types=[
            pltpu.VMEM((ROWS_PER_W, IDX_W), jnp.int32),
            pltpu.VMEM((CHUNK, EMB), jnp.float32),
            pltpu.VMEM((CHUNK, EMB), jnp.float32),
            pltpu.SemaphoreType.DMA,
            pltpu.SemaphoreType.DMA,
            pltpu.SemaphoreType.DMA,
        ],
    )
    def emb_kernel(items_hbm, table_hbm, out_hbm, idx_v, rows0, rows1,
                   gsem, wsem0, wsem1):
        wid = lax.axis_index("s") * 2 + lax.axis_index("c")
        row_base = wid * ROWS_PER_W
        out_base0 = row_base * IDX_W
        pltpu.sync_copy(items_hbm.at[pl.ds(row_base, ROWS_PER_W)], idx_v)

        bufs = ((rows0, wsem0), (rows1, wsem1))

        def superstep(t):
            for b in range(2):
                rows_v, wsem = bufs[b]
                g = t * 2 + b

                @pl.when(t > 0)
                def _():
                    pltpu.make_async_copy(
                        rows_v,
                        out_hbm.at[pl.ds(out_base0 + (g - 2) * CHUNK, CHUNK)],
                        wsem,
                    ).wait()

                handles = []
                for j in range(STREAMS_PER_CHUNK):
                    handles.append(pltpu.async_copy(
                        table_hbm.at[idx_v.at[g * STREAMS_PER_CHUNK + j]],
                        rows_v.at[pl.ds(j * IDX_W, IDX_W)],
                        gsem,
                    ))
                for h in handles:
                    h.wait()
                pltpu.async_copy(
                    rows_v,
                    out_hbm.at[pl.ds(out_base0 + g * CHUNK, CHUNK)],
                    wsem,
                )

        pl.loop(0, CHUNKS // 2)(superstep)

        for b in range(2):
            rows_v, wsem = bufs[b]
            g = CHUNKS - 2 + b
            pltpu.make_async_copy(
                rows_v,
                out_hbm.at[pl.ds(out_base0 + g * CHUNK, CHUNK)],
                wsem,
            ).wait()

    return emb_kernel


_emb_kernel = _make_kernel()


def kernel(items, weight):
    items2d = items.reshape(B // IDX_W, IDX_W).astype(jnp.int32)
    out = _emb_kernel(items2d, weight)
    return out.reshape(BATCH, HIST, EMB)


# speedup vs baseline: 1.8771x; 1.0025x over previous
import functools

import jax
import jax.numpy as jnp
from jax import lax
from jax.experimental import pallas as pl
from jax.experimental.pallas import tpu as pltpu
from jax.experimental.pallas import tpu_sc as plsc

VOCAB = 1000000
EMB = 64
BATCH = 16384
HIST = 50

B = BATCH * HIST
IDX_W = 128
NW = 32
ROWS_PER_W = B // (NW * IDX_W)
STREAMS_PER_CHUNK = 5
CHUNK = STREAMS_PER_CHUNK * IDX_W
CHUNKS = ROWS_PER_W // STREAMS_PER_CHUNK


def _make_kernel():
    mesh = plsc.VectorSubcoreMesh(core_axis_name="c", subcore_axis_name="s")

    @functools.partial(
        pl.kernel,
        mesh=mesh,
        compiler_params=pltpu.CompilerParams(use_tc_tiling_on_sc=False),
        out_type=jax.ShapeDtypeStruct((B, EMB), jnp.float32),
        scratch_types=[
            pltpu.VMEM((ROWS_PER_W, IDX_W), jnp.int32),
            pltpu.VMEM((CHUNK, EMB), jnp.float32),
            pltpu.VMEM((CHUNK, EMB), jnp.float32),
            pltpu.SemaphoreType.DMA,
            pltpu.SemaphoreType.DMA,
            pltpu.SemaphoreType.DMA,
            pltpu.SemaphoreType.DMA,
        ],
    )
    def emb_kernel(items_hbm, table_hbm, out_hbm, idx_v, rows0, rows1,
                   gsem0, gsem1, wsem0, wsem1):
        wid = lax.axis_index("s") * 2 + lax.axis_index("c")
        row_base = wid * ROWS_PER_W
        out_base0 = row_base * IDX_W
        pltpu.sync_copy(items_hbm.at[pl.ds(row_base, ROWS_PER_W)], idx_v)

        bufs = ((rows0, gsem0, wsem0), (rows1, gsem1, wsem1))

        def fire_gathers(g, rows_v, gsem):
            for j in range(STREAMS_PER_CHUNK):
                pltpu.async_copy(
                    table_hbm.at[idx_v.at[g * STREAMS_PER_CHUNK + j]],
                    rows_v.at[pl.ds(j * IDX_W, IDX_W)],
                    gsem,
                )

        def drain_gathers(g, rows_v, gsem):
            for j in range(STREAMS_PER_CHUNK):
                pltpu.make_async_copy(
                    table_hbm.at[idx_v.at[g * STREAMS_PER_CHUNK + j]],
                    rows_v.at[pl.ds(j * IDX_W, IDX_W)],
                    gsem,
                ).wait()

        fire_gathers(0, rows0, gsem0)

        def superstep(t):
            for b in range(2):
                g = t * 2 + b
                nb = 1 - b
                rows_b, gsem_b, wsem_b = bufs[b]
                rows_nb, gsem_nb, wsem_nb = bufs[nb]

                @pl.when(g > 0)
                def _():
                    pltpu.make_async_copy(
                        rows_nb,
                        out_hbm.at[pl.ds(out_base0 + (g - 1) * CHUNK, CHUNK)],
                        wsem_nb,
                    ).wait()

                @pl.when(g + 1 < CHUNKS)
                def _():
                    fire_gathers(g + 1, rows_nb, gsem_nb)

                drain_gathers(g, rows_b, gsem_b)
                pltpu.async_copy(
                    rows_b,
                    out_hbm.at[pl.ds(out_base0 + g * CHUNK, CHUNK)],
                    wsem_b,
                )

        pl.loop(0, CHUNKS // 2)(superstep)

        pltpu.make_async_copy(
            rows1,
            out_hbm.at[pl.ds(out_base0 + (CHUNKS - 1) * CHUNK, CHUNK)],
            wsem1,
        ).wait()

    return emb_kernel


_emb_kernel = _make_kernel()


def kernel(items, weight):
    items2d = items.reshape(B // IDX_W, IDX_W).astype(jnp.int32)
    out = _emb_kernel(items2d, weight)
    return out.reshape(BATCH, HIST, EMB)
